# initial kernel scaffold (unmeasured)
import functools

import jax
import jax.numpy as jnp
from jax import lax
from jax.experimental import pallas as pl
from jax.experimental.pallas import tpu as pltpu

N_DEV = 4
N_EXPERTS = 32
CAP = 51
CAP_PAD = 64
E_LOCAL = N_EXPERTS // N_DEV


def _moe_allgather_pallas(x_disp, expert_W):
    _, _, d_model = x_disp.shape
    _, _, h_out = expert_W.shape

    def body(xd_ref, w_ref, out_ref, send_sems, recv_sems):
        my = lax.axis_index("i")
        left = lax.rem(my + (N_DEV - 1), N_DEV)
        right = lax.rem(my + 1, N_DEV)

        barrier_sem = pltpu.get_barrier_semaphore()
        for nbr in [left, right]:
            pl.semaphore_signal(
                barrier_sem, inc=1,
                device_id=(nbr,), device_id_type=pl.DeviceIdType.MESH,
            )
        pl.semaphore_wait(barrier_sem, 2)

        for e in range(E_LOCAL):
            res = jnp.dot(
                xd_ref[e],
                w_ref[e].astype(jnp.bfloat16),
                preferred_element_type=jnp.float32,
            )
            out_ref[pl.ds(my * E_LOCAL + e, 1)] = res.astype(jnp.bfloat16)[None]

        for h in range(N_DEV - 1):
            chunk = lax.rem(my - h + N_DEV, N_DEV)
            rdma = pltpu.make_async_remote_copy(
                src_ref=out_ref.at[pl.ds(chunk * E_LOCAL, E_LOCAL)],
                dst_ref=out_ref.at[pl.ds(chunk * E_LOCAL, E_LOCAL)],
                send_sem=send_sems.at[h],
                recv_sem=recv_sems.at[h],
                device_id=(right,),
                device_id_type=pl.DeviceIdType.MESH,
            )
            rdma.start()
            rdma.wait()

    return pl.pallas_call(
        body,
        out_shape=jax.ShapeDtypeStruct((N_EXPERTS, CAP_PAD, h_out), jnp.bfloat16),
        in_specs=[
            pl.BlockSpec(memory_space=pltpu.VMEM),
            pl.BlockSpec(memory_space=pltpu.VMEM),
        ],
        out_specs=pl.BlockSpec(memory_space=pltpu.VMEM),
        scratch_shapes=[
            pltpu.SemaphoreType.DMA((N_DEV - 1,)),
            pltpu.SemaphoreType.DMA((N_DEV - 1,)),
        ],
        compiler_params=pltpu.CompilerParams(
            collective_id=0,
            vmem_limit_bytes=100 * 1024 * 1024,
        ),
    )(x_disp, expert_W)


def kernel(x, router_W, route_idx, expert_W):
    n_tokens, d_model = x.shape
    del router_W
    my = lax.axis_index("i")
    tok_per_dev = n_tokens // N_DEV

    e = route_idx[:, 0]
    onehot = (e[:, None] == jnp.arange(N_EXPERTS, dtype=e.dtype)[None, :])
    cum = jnp.cumsum(onehot.astype(jnp.int32), axis=0)
    pos = jnp.take_along_axis(cum, e[:, None].astype(jnp.int32), axis=1)[:, 0] - 1
    kept = pos < CAP
    slot = e * CAP_PAD + pos
    n_rows = N_EXPERTS * CAP_PAD
    slot_safe = jnp.where(kept, slot, n_rows)
    src_token = (
        jnp.zeros((n_rows,), jnp.int32)
        .at[slot_safe]
        .set(jnp.arange(n_tokens, dtype=jnp.int32), mode="drop")
    )

    rows_per_dev = E_LOCAL * CAP_PAD
    local_src = lax.dynamic_slice(src_token, (my * rows_per_dev,), (rows_per_dev,))
    x_disp = x[local_src].astype(jnp.bfloat16).reshape(E_LOCAL, CAP_PAD, d_model)

    table = _moe_allgather_pallas(x_disp, expert_W)
    table_flat = table.reshape(n_rows, -1)

    my_slot = lax.dynamic_slice(slot, (my * tok_per_dev,), (tok_per_dev,))
    my_kept = lax.dynamic_slice(kept, (my * tok_per_dev,), (tok_per_dev,))
    rows = table_flat[jnp.clip(my_slot, 0, n_rows - 1)]
    return jnp.where(my_kept[:, None], rows, 0).astype(jnp.float32)


# baseline (device time: 82462 ns/iter reference)
import jax
import jax.numpy as jnp
from jax import lax
from jax.experimental import pallas as pl
from jax.experimental.pallas import tpu as pltpu

N_DEV = 4
N_EXPERTS = 32
CAP = 51
CAP_PAD = 64
E_LOCAL = N_EXPERTS // N_DEV
ROWS_PER_DEV = E_LOCAL * CAP_PAD
N_ROWS = N_EXPERTS * CAP_PAD
DROP_SENTINEL = 3000


def _moe_pallas(x, slot_row, slot_col, expert_W, tok_per_dev):
    n_tokens, d_model = x.shape
    _, _, h_out = expert_W.shape

    def body(x_ref, srow_ref, scol_ref, w_ref, out_ref, table, xd,
             send_sems, recv_sems):
        my = lax.axis_index("i")
        left = lax.rem(my + (N_DEV - 1), N_DEV)
        right = lax.rem(my + 1, N_DEV)
        base = my * ROWS_PER_DEV

        barrier_sem = pltpu.get_barrier_semaphore()
        for nbr in [left, right]:
            pl.semaphore_signal(
                barrier_sem, inc=1,
                device_id=(nbr,), device_id_type=pl.DeviceIdType.MESH,
            )
        pl.semaphore_wait(barrier_sem, 2)

        row_ids = base + lax.broadcasted_iota(
            jnp.int32, (ROWS_PER_DEV, n_tokens), 0
        )
        disp = (srow_ref[...] == row_ids).astype(jnp.bfloat16)
        xd[...] = jnp.dot(
            disp, x_ref[...].astype(jnp.bfloat16),
            preferred_element_type=jnp.float32,
        ).astype(jnp.bfloat16)

        for e in range(E_LOCAL):
            res = jnp.dot(
                xd[e * CAP_PAD:(e + 1) * CAP_PAD],
                w_ref[e].astype(jnp.bfloat16),
                preferred_element_type=jnp.float32,
            )
            table[pl.ds(base + e * CAP_PAD, CAP_PAD), :] = res.astype(
                jnp.bfloat16
            )

        for h in range(N_DEV - 1):
            chunk = lax.rem(my - h + N_DEV, N_DEV)
            rdma = pltpu.make_async_remote_copy(
                src_ref=table.at[pl.ds(chunk * ROWS_PER_DEV, ROWS_PER_DEV)],
                dst_ref=table.at[pl.ds(chunk * ROWS_PER_DEV, ROWS_PER_DEV)],
                send_sem=send_sems.at[h],
                recv_sem=recv_sems.at[h],
                device_id=(right,),
                device_id_type=pl.DeviceIdType.MESH,
            )
            rdma.start()
            rdma.wait()

        my_slot = scol_ref[pl.ds(my * tok_per_dev, tok_per_dev), :]
        col_ids = lax.broadcasted_iota(
            jnp.int32, (tok_per_dev, N_ROWS), 1
        )
        comb = (my_slot == col_ids).astype(jnp.bfloat16)
        out_ref[...] = jnp.dot(
            comb, table[...], preferred_element_type=jnp.float32
        )

    return pl.pallas_call(
        body,
        out_shape=jax.ShapeDtypeStruct((tok_per_dev, h_out), jnp.float32),
        in_specs=[
            pl.BlockSpec(memory_space=pltpu.VMEM),
            pl.BlockSpec(memory_space=pltpu.VMEM),
            pl.BlockSpec(memory_space=pltpu.VMEM),
            pl.BlockSpec(memory_space=pltpu.VMEM),
        ],
        out_specs=pl.BlockSpec(memory_space=pltpu.VMEM),
        scratch_shapes=[
            pltpu.VMEM((N_ROWS, h_out), jnp.bfloat16),
            pltpu.VMEM((ROWS_PER_DEV, d_model), jnp.bfloat16),
            pltpu.SemaphoreType.DMA((N_DEV - 1,)),
            pltpu.SemaphoreType.DMA((N_DEV - 1,)),
        ],
        compiler_params=pltpu.CompilerParams(
            collective_id=0,
            vmem_limit_bytes=100 * 1024 * 1024,
        ),
    )(x, slot_row, slot_col, expert_W)


def kernel(x, router_W, route_idx, expert_W):
    n_tokens, _ = x.shape
    del router_W
    tok_per_dev = n_tokens // N_DEV

    e = route_idx[:, 0]
    onehot = (e[:, None] == jnp.arange(N_EXPERTS, dtype=e.dtype)[None, :])
    oh32 = onehot.astype(jnp.int32)
    cum = jnp.cumsum(oh32, axis=0)
    pos = jnp.sum(cum * oh32, axis=1) - 1
    kept = pos < CAP
    slot = jnp.where(kept, e * CAP_PAD + pos, DROP_SENTINEL).astype(jnp.int32)

    return _moe_pallas(
        x, slot[None, :], slot[:, None], expert_W, tok_per_dev
    )


# device time: 71188 ns/iter; 1.1584x vs baseline; 1.1584x over previous
import os

import jax
import jax.numpy as jnp
from jax import lax
from jax.experimental import pallas as pl
from jax.experimental.pallas import tpu as pltpu

N_DEV = 4
N_EXPERTS = 32
CAP = 51
CAP_PAD = 64
E_LOCAL = N_EXPERTS // N_DEV
ROWS_PER_DEV = E_LOCAL * CAP_PAD
HALF = ROWS_PER_DEV // 2
N_ROWS = N_EXPERTS * CAP_PAD
DROP_SENTINEL = 3000


def _moe_pallas(x, slot_row, slot_col, expert_W, tok_per_dev):
    n_tokens, d_model = x.shape
    _, _, h_out = expert_W.shape

    def body(x_ref, srow_ref, scol_ref, w_hbm, out_ref, table, xd, wbuf,
             w_sems, send_cw, recv_cw, send_ccw, recv_ccw):
        my = lax.axis_index("i")
        left = lax.rem(my + (N_DEV - 1), N_DEV)
        right = lax.rem(my + 1, N_DEV)
        base = my * ROWS_PER_DEV

        barrier_sem = pltpu.get_barrier_semaphore()
        for nbr in [left, right]:
            pl.semaphore_signal(
                barrier_sem, inc=1,
                device_id=(nbr,), device_id_type=pl.DeviceIdType.MESH,
            )
        pl.semaphore_wait(barrier_sem, 2)

        def w_copy(e):
            return pltpu.make_async_copy(
                w_hbm.at[e], wbuf.at[e % 2], w_sems.at[e % 2]
            )

        w_copy(0).start()

        row_ids = base + lax.broadcasted_iota(
            jnp.int32, (ROWS_PER_DEV, n_tokens), 0
        )
        disp = (srow_ref[...] == row_ids).astype(jnp.bfloat16)
        xd[...] = jnp.dot(
            disp, x_ref[...].astype(jnp.bfloat16),
            preferred_element_type=jnp.float32,
        ).astype(jnp.bfloat16)

        for e in range(E_LOCAL):
            w_copy(e).wait()
            if e + 1 < E_LOCAL:
                w_copy(e + 1).start()
            res = jnp.dot(
                xd[e * CAP_PAD:(e + 1) * CAP_PAD],
                wbuf[e % 2].astype(jnp.bfloat16),
                preferred_element_type=jnp.float32,
            )
            table[pl.ds(base + e * CAP_PAD, CAP_PAD), :] = res.astype(
                jnp.bfloat16
            )

        n_hops = 0 if os.environ.get("KERNEL_NO_RING") else N_DEV - 1
        for h in range(n_hops):
            c_cw = lax.rem(my - h + N_DEV, N_DEV)
            c_ccw = lax.rem(my + h, N_DEV)
            rdma_cw = pltpu.make_async_remote_copy(
                src_ref=table.at[pl.ds(c_cw * ROWS_PER_DEV, HALF)],
                dst_ref=table.at[pl.ds(c_cw * ROWS_PER_DEV, HALF)],
                send_sem=send_cw.at[h],
                recv_sem=recv_cw.at[h],
                device_id=(right,),
                device_id_type=pl.DeviceIdType.MESH,
            )
            rdma_ccw = pltpu.make_async_remote_copy(
                src_ref=table.at[pl.ds(c_ccw * ROWS_PER_DEV + HALF, HALF)],
                dst_ref=table.at[pl.ds(c_ccw * ROWS_PER_DEV + HALF, HALF)],
                send_sem=send_ccw.at[h],
                recv_sem=recv_ccw.at[h],
                device_id=(left,),
                device_id_type=pl.DeviceIdType.MESH,
            )
            rdma_cw.start()
            rdma_ccw.start()
            rdma_cw.wait()
            rdma_ccw.wait()

        my_slot = scol_ref[pl.ds(my * tok_per_dev, tok_per_dev), :]
        col_ids = lax.broadcasted_iota(
            jnp.int32, (tok_per_dev, N_ROWS), 1
        )
        comb = (my_slot == col_ids).astype(jnp.bfloat16)
        out_ref[...] = jnp.dot(
            comb, table[...], preferred_element_type=jnp.float32
        )

    return pl.pallas_call(
        body,
        out_shape=jax.ShapeDtypeStruct((tok_per_dev, h_out), jnp.float32),
        in_specs=[
            pl.BlockSpec(memory_space=pltpu.VMEM),
            pl.BlockSpec(memory_space=pltpu.VMEM),
            pl.BlockSpec(memory_space=pltpu.VMEM),
            pl.BlockSpec(memory_space=pltpu.MemorySpace.HBM),
        ],
        out_specs=pl.BlockSpec(memory_space=pltpu.VMEM),
        scratch_shapes=[
            pltpu.VMEM((N_ROWS, h_out), jnp.bfloat16),
            pltpu.VMEM((ROWS_PER_DEV, d_model), jnp.bfloat16),
            pltpu.VMEM((2, d_model, h_out), jnp.float32),
            pltpu.SemaphoreType.DMA((2,)),
            pltpu.SemaphoreType.DMA((N_DEV - 1,)),
            pltpu.SemaphoreType.DMA((N_DEV - 1,)),
            pltpu.SemaphoreType.DMA((N_DEV - 1,)),
            pltpu.SemaphoreType.DMA((N_DEV - 1,)),
        ],
        compiler_params=pltpu.CompilerParams(
            collective_id=0,
            vmem_limit_bytes=100 * 1024 * 1024,
        ),
    )(x, slot_row, slot_col, expert_W)


def kernel(x, router_W, route_idx, expert_W):
    n_tokens, _ = x.shape
    del router_W
    tok_per_dev = n_tokens // N_DEV

    e = route_idx[:, 0]
    onehot = (e[:, None] == jnp.arange(N_EXPERTS, dtype=e.dtype)[None, :])
    oh32 = onehot.astype(jnp.int32)
    cum = jnp.cumsum(oh32, axis=0)
    pos = jnp.sum(cum * oh32, axis=1) - 1
    kept = pos < CAP
    slot = jnp.where(kept, e * CAP_PAD + pos, DROP_SENTINEL).astype(jnp.int32)

    return _moe_pallas(
        x, slot[None, :], slot[:, None], expert_W, tok_per_dev
    )


# device time: 59475 ns/iter; 1.3865x vs baseline; 1.1969x over previous
import os

import jax
import jax.numpy as jnp
from jax import lax
from jax.experimental import pallas as pl
from jax.experimental.pallas import tpu as pltpu

N_DEV = 4
N_EXPERTS = 32
CAP = 51
CAP_PAD = 64
E_LOCAL = N_EXPERTS // N_DEV
ROWS_PER_DEV = E_LOCAL * CAP_PAD
HALF = ROWS_PER_DEV // 2
N_ROWS = N_EXPERTS * CAP_PAD
BLK = 256
DROP_SENTINEL = 3000


def _moe_pallas(x, rt_col, rt_row, expert_W, tok_per_dev):
    n_tokens, d_model = x.shape
    _, _, h_out = expert_W.shape
    n_blk = n_tokens // BLK

    def body(x_ref, rtc_ref, rtr_ref, w_hbm, out_ref, table, xd,
             slot_col, slot_row, wbuf, w_sems,
             send_cw, recv_cw, send_ccw, recv_ccw):
        my = lax.axis_index("i")
        left = lax.rem(my + (N_DEV - 1), N_DEV)
        right = lax.rem(my + 1, N_DEV)
        base = my * ROWS_PER_DEV

        barrier_sem = pltpu.get_barrier_semaphore()
        for nbr in [left, right]:
            pl.semaphore_signal(
                barrier_sem, inc=1,
                device_id=(nbr,), device_id_type=pl.DeviceIdType.MESH,
            )
        pl.semaphore_wait(barrier_sem, 2)

        def w_copy(e):
            return pltpu.make_async_copy(
                w_hbm.at[e], wbuf.at[e % 2], w_sems.at[e % 2]
            )

        w_copy(0).start()

        ir = lax.broadcasted_iota(jnp.int32, (BLK, BLK), 0)
        ic = lax.broadcasted_iota(jnp.int32, (BLK, BLK), 1)
        tril = (ir >= ic).astype(jnp.bfloat16)
        triu = (ir <= ic).astype(jnp.bfloat16)

        lane32 = lax.broadcasted_iota(jnp.int32, (BLK, N_EXPERTS), 1)
        prefix = jnp.zeros((1, N_EXPERTS), jnp.float32)
        for b in range(n_blk):
            e_b = rtc_ref[b * BLK:(b + 1) * BLK, :]
            ohm = e_b == lane32
            cum_b = jnp.dot(
                tril, ohm.astype(jnp.bfloat16),
                preferred_element_type=jnp.float32,
            ) + prefix
            pos_b = jnp.sum(
                cum_b * ohm.astype(jnp.float32), axis=1, keepdims=True
            ) - 1.0
            slot_f = jnp.where(
                pos_b < float(CAP),
                e_b.astype(jnp.float32) * float(CAP_PAD) + pos_b,
                float(DROP_SENTINEL),
            )
            slot_col[b * BLK:(b + 1) * BLK, :] = slot_f.astype(jnp.int32)
            prefix = cum_b[BLK - 1:BLK, :]

        sub32 = lax.broadcasted_iota(jnp.int32, (N_EXPERTS, BLK), 0)
        prefr = jnp.zeros((N_EXPERTS, 1), jnp.float32)
        for b in range(n_blk):
            e_rb = rtr_ref[:, b * BLK:(b + 1) * BLK]
            ohm = sub32 == e_rb
            cum_rb = jnp.dot(
                ohm.astype(jnp.bfloat16), triu,
                preferred_element_type=jnp.float32,
            ) + prefr
            pos_rb = jnp.sum(
                cum_rb * ohm.astype(jnp.float32), axis=0, keepdims=True
            ) - 1.0
            slot_rf = jnp.where(
                pos_rb < float(CAP),
                e_rb.astype(jnp.float32) * float(CAP_PAD) + pos_rb,
                float(DROP_SENTINEL),
            )
            slot_row[:, b * BLK:(b + 1) * BLK] = slot_rf.astype(jnp.int32)
            prefr = cum_rb[:, BLK - 1:BLK]

        row_ids = base + lax.broadcasted_iota(
            jnp.int32, (ROWS_PER_DEV, n_tokens), 0
        )
        disp = (slot_row[...] == row_ids).astype(jnp.bfloat16)
        xd[...] = jnp.dot(
            disp, x_ref[...].astype(jnp.bfloat16),
            preferred_element_type=jnp.float32,
        ).astype(jnp.bfloat16)

        for e in range(E_LOCAL):
            w_copy(e).wait()
            if e + 1 < E_LOCAL:
                w_copy(e + 1).start()
            res = jnp.dot(
                xd[e * CAP_PAD:(e + 1) * CAP_PAD],
                wbuf[e % 2].astype(jnp.bfloat16),
                preferred_element_type=jnp.float32,
            )
            table[pl.ds(base + e * CAP_PAD, CAP_PAD), :] = res.astype(
                jnp.bfloat16
            )

        n_hops = 0 if os.environ.get("KERNEL_NO_RING") else N_DEV - 1
        for h in range(n_hops):
            c_cw = lax.rem(my - h + N_DEV, N_DEV)
            c_ccw = lax.rem(my + h, N_DEV)
            rdma_cw = pltpu.make_async_remote_copy(
                src_ref=table.at[pl.ds(c_cw * ROWS_PER_DEV, HALF)],
                dst_ref=table.at[pl.ds(c_cw * ROWS_PER_DEV, HALF)],
                send_sem=send_cw.at[h],
                recv_sem=recv_cw.at[h],
                device_id=(right,),
                device_id_type=pl.DeviceIdType.MESH,
            )
            rdma_ccw = pltpu.make_async_remote_copy(
                src_ref=table.at[pl.ds(c_ccw * ROWS_PER_DEV + HALF, HALF)],
                dst_ref=table.at[pl.ds(c_ccw * ROWS_PER_DEV + HALF, HALF)],
                send_sem=send_ccw.at[h],
                recv_sem=recv_ccw.at[h],
                device_id=(left,),
                device_id_type=pl.DeviceIdType.MESH,
            )
            rdma_cw.start()
            rdma_ccw.start()
            rdma_cw.wait()
            rdma_ccw.wait()

        my_slot = slot_col[pl.ds(my * tok_per_dev, tok_per_dev), :]
        col_ids = lax.broadcasted_iota(
            jnp.int32, (tok_per_dev, N_ROWS), 1
        )
        comb = (my_slot == col_ids).astype(jnp.bfloat16)
        out_ref[...] = jnp.dot(
            comb, table[...], preferred_element_type=jnp.float32
        )

    return pl.pallas_call(
        body,
        out_shape=jax.ShapeDtypeStruct((tok_per_dev, h_out), jnp.float32),
        in_specs=[
            pl.BlockSpec(memory_space=pltpu.VMEM),
            pl.BlockSpec(memory_space=pltpu.VMEM),
            pl.BlockSpec(memory_space=pltpu.VMEM),
            pl.BlockSpec(memory_space=pltpu.MemorySpace.HBM),
        ],
        out_specs=pl.BlockSpec(memory_space=pltpu.VMEM),
        scratch_shapes=[
            pltpu.VMEM((N_ROWS, h_out), jnp.bfloat16),
            pltpu.VMEM((ROWS_PER_DEV, d_model), jnp.bfloat16),
            pltpu.VMEM((n_tokens, 1), jnp.int32),
            pltpu.VMEM((1, n_tokens), jnp.int32),
            pltpu.VMEM((2, d_model, h_out), jnp.float32),
            pltpu.SemaphoreType.DMA((2,)),
            pltpu.SemaphoreType.DMA((N_DEV - 1,)),
            pltpu.SemaphoreType.DMA((N_DEV - 1,)),
            pltpu.SemaphoreType.DMA((N_DEV - 1,)),
            pltpu.SemaphoreType.DMA((N_DEV - 1,)),
        ],
        compiler_params=pltpu.CompilerParams(
            collective_id=0,
            vmem_limit_bytes=100 * 1024 * 1024,
        ),
    )(x, rt_col, rt_row, expert_W)


def kernel(x, router_W, route_idx, expert_W):
    n_tokens, _ = x.shape
    del router_W
    tok_per_dev = n_tokens // N_DEV
    rt_col = route_idx.astype(jnp.int32)
    rt_row = jnp.transpose(rt_col)
    return _moe_pallas(x, rt_col, rt_row, expert_W, tok_per_dev)


# device time: 46339 ns/iter; 1.7795x vs baseline; 1.2835x over previous
import os

import jax
import jax.numpy as jnp
from jax import lax
from jax.experimental import pallas as pl
from jax.experimental.pallas import tpu as pltpu

N_DEV = 4
N_EXPERTS = 32
CAP = 51
CAP_PAD = 64
E_LOCAL = N_EXPERTS // N_DEV
ROWS_PER_DEV = E_LOCAL * CAP_PAD
N_SUB = E_LOCAL // 2
N_WBUF = 4
N_ROWS = N_EXPERTS * CAP_PAD
BLK = 256
DROP_SENTINEL = 3000


def _moe_pallas(x, rt_col, rt_row, expert_W, tok_per_dev):
    n_tokens, d_model = x.shape
    _, _, h_out = expert_W.shape
    n_blk = n_tokens // BLK

    def body(x_ref, rtc_ref, rtr_ref, w_hbm, out_ref, table, xd,
             slot_col, slot_row, wbuf, w_sems,
             send_cw, recv_cw, send_ccw, recv_ccw):
        my = lax.axis_index("i")
        left = lax.rem(my + (N_DEV - 1), N_DEV)
        right = lax.rem(my + 1, N_DEV)
        base = my * ROWS_PER_DEV
        pipe = not os.environ.get("KERNEL_NO_RING")

        barrier_sem = pltpu.get_barrier_semaphore()
        for nbr in [left, right]:
            pl.semaphore_signal(
                barrier_sem, inc=1,
                device_id=(nbr,), device_id_type=pl.DeviceIdType.MESH,
            )
        pl.semaphore_wait(barrier_sem, 2)

        def w_copy(e):
            return pltpu.make_async_copy(
                w_hbm.at[e], wbuf.at[e % N_WBUF], w_sems.at[e % N_WBUF]
            )

        for e in range(N_WBUF):
            w_copy(e).start()

        def mk_cw(h, c, s):
            r0 = c * ROWS_PER_DEV + s * CAP_PAD
            return pltpu.make_async_remote_copy(
                src_ref=table.at[pl.ds(r0, CAP_PAD)],
                dst_ref=table.at[pl.ds(r0, CAP_PAD)],
                send_sem=send_cw.at[h, s],
                recv_sem=recv_cw.at[h, s],
                device_id=(right,),
                device_id_type=pl.DeviceIdType.MESH,
            )

        def mk_ccw(h, c, s):
            r0 = c * ROWS_PER_DEV + (N_SUB + s) * CAP_PAD
            return pltpu.make_async_remote_copy(
                src_ref=table.at[pl.ds(r0, CAP_PAD)],
                dst_ref=table.at[pl.ds(r0, CAP_PAD)],
                send_sem=send_ccw.at[h, s],
                recv_sem=recv_ccw.at[h, s],
                device_id=(left,),
                device_id_type=pl.DeviceIdType.MESH,
            )

        ir = lax.broadcasted_iota(jnp.int32, (BLK, BLK), 0)
        ic = lax.broadcasted_iota(jnp.int32, (BLK, BLK), 1)
        tril = (ir >= ic).astype(jnp.bfloat16)
        triu = (ir <= ic).astype(jnp.bfloat16)

        lane32 = lax.broadcasted_iota(jnp.int32, (BLK, N_EXPERTS), 1)
        prefix = jnp.zeros((1, N_EXPERTS), jnp.float32)
        for b in range(n_blk):
            e_b = rtc_ref[b * BLK:(b + 1) * BLK, :]
            ohm = e_b == lane32
            cum_b = jnp.dot(
                tril, ohm.astype(jnp.bfloat16),
                preferred_element_type=jnp.float32,
            ) + prefix
            pos_b = jnp.sum(
                cum_b * ohm.astype(jnp.float32), axis=1, keepdims=True
            ) - 1.0
            slot_f = jnp.where(
                pos_b < float(CAP),
                e_b.astype(jnp.float32) * float(CAP_PAD) + pos_b,
                float(DROP_SENTINEL),
            )
            slot_col[b * BLK:(b + 1) * BLK, :] = slot_f.astype(jnp.int32)
            prefix = cum_b[BLK - 1:BLK, :]

        sub32 = lax.broadcasted_iota(jnp.int32, (N_EXPERTS, BLK), 0)
        prefr = jnp.zeros((N_EXPERTS, 1), jnp.float32)
        for b in range(n_blk):
            e_rb = rtr_ref[:, b * BLK:(b + 1) * BLK]
            ohm = sub32 == e_rb
            cum_rb = jnp.dot(
                ohm.astype(jnp.bfloat16), triu,
                preferred_element_type=jnp.float32,
            ) + prefr
            pos_rb = jnp.sum(
                cum_rb * ohm.astype(jnp.float32), axis=0, keepdims=True
            ) - 1.0
            slot_rf = jnp.where(
                pos_rb < float(CAP),
                e_rb.astype(jnp.float32) * float(CAP_PAD) + pos_rb,
                float(DROP_SENTINEL),
            )
            slot_row[:, b * BLK:(b + 1) * BLK] = slot_rf.astype(jnp.int32)
            prefr = cum_rb[:, BLK - 1:BLK]

        row_ids = base + lax.broadcasted_iota(
            jnp.int32, (ROWS_PER_DEV, n_tokens), 0
        )
        disp = (slot_row[...] == row_ids).astype(jnp.bfloat16)
        xd[...] = jnp.dot(
            disp, x_ref[...].astype(jnp.bfloat16),
            preferred_element_type=jnp.float32,
        ).astype(jnp.bfloat16)

        for e in range(E_LOCAL):
            w_copy(e).wait()
            res = jnp.dot(
                xd[e * CAP_PAD:(e + 1) * CAP_PAD],
                wbuf[e % N_WBUF].astype(jnp.bfloat16),
                preferred_element_type=jnp.float32,
            )
            table[pl.ds(base + e * CAP_PAD, CAP_PAD), :] = res.astype(
                jnp.bfloat16
            )
            if e + N_WBUF < E_LOCAL:
                w_copy(e + N_WBUF).start()
            if pipe:
                if e < N_SUB:
                    mk_cw(0, my, e).start()
                else:
                    mk_ccw(0, my, e - N_SUB).start()

        if pipe:
            for h in range(1, N_DEV - 1):
                c_cw = lax.rem(my - h + N_DEV, N_DEV)
                c_ccw = lax.rem(my + h, N_DEV)
                for s in range(N_SUB):
                    mk_cw(h - 1, c_cw, s).wait_recv()
                    mk_cw(h, c_cw, s).start()
                    mk_ccw(h - 1, c_ccw, s).wait_recv()
                    mk_ccw(h, c_ccw, s).start()
            h_last = N_DEV - 2
            c_cw = lax.rem(my + 1, N_DEV)
            c_ccw = lax.rem(my - 1 + N_DEV, N_DEV)
            for s in range(N_SUB):
                mk_cw(h_last, c_cw, s).wait_recv()
                mk_ccw(h_last, c_ccw, s).wait_recv()
            for h in range(N_DEV - 1):
                c_cw = lax.rem(my - h + N_DEV, N_DEV)
                c_ccw = lax.rem(my + h, N_DEV)
                for s in range(N_SUB):
                    mk_cw(h, c_cw, s).wait_send()
                    mk_ccw(h, c_ccw, s).wait_send()

        my_slot = slot_col[pl.ds(my * tok_per_dev, tok_per_dev), :]
        col_ids = lax.broadcasted_iota(
            jnp.int32, (tok_per_dev, N_ROWS), 1
        )
        comb = (my_slot == col_ids).astype(jnp.bfloat16)
        out_ref[...] = jnp.dot(
            comb, table[...], preferred_element_type=jnp.float32
        )

    return pl.pallas_call(
        body,
        out_shape=jax.ShapeDtypeStruct((tok_per_dev, h_out), jnp.float32),
        in_specs=[
            pl.BlockSpec(memory_space=pltpu.VMEM),
            pl.BlockSpec(memory_space=pltpu.VMEM),
            pl.BlockSpec(memory_space=pltpu.VMEM),
            pl.BlockSpec(memory_space=pltpu.MemorySpace.HBM),
        ],
        out_specs=pl.BlockSpec(memory_space=pltpu.VMEM),
        scratch_shapes=[
            pltpu.VMEM((N_ROWS, h_out), jnp.bfloat16),
            pltpu.VMEM((ROWS_PER_DEV, d_model), jnp.bfloat16),
            pltpu.VMEM((n_tokens, 1), jnp.int32),
            pltpu.VMEM((1, n_tokens), jnp.int32),
            pltpu.VMEM((N_WBUF, d_model, h_out), jnp.float32),
            pltpu.SemaphoreType.DMA((N_WBUF,)),
            pltpu.SemaphoreType.DMA((N_DEV - 1, N_SUB)),
            pltpu.SemaphoreType.DMA((N_DEV - 1, N_SUB)),
            pltpu.SemaphoreType.DMA((N_DEV - 1, N_SUB)),
            pltpu.SemaphoreType.DMA((N_DEV - 1, N_SUB)),
        ],
        compiler_params=pltpu.CompilerParams(
            collective_id=0,
            vmem_limit_bytes=100 * 1024 * 1024,
        ),
    )(x, rt_col, rt_row, expert_W)


def kernel(x, router_W, route_idx, expert_W):
    n_tokens, _ = x.shape
    del router_W
    tok_per_dev = n_tokens // N_DEV
    rt_col = route_idx.astype(jnp.int32)
    rt_row = jnp.transpose(rt_col)
    return _moe_pallas(x, rt_col, rt_row, expert_W, tok_per_dev)


# device time: 45105 ns/iter; 1.8282x vs baseline; 1.0274x over previous
import os

import jax
import jax.numpy as jnp
from jax import lax
from jax.experimental import pallas as pl
from jax.experimental.pallas import tpu as pltpu

N_DEV = 4
N_EXPERTS = 32
CAP = 51
CAP_PAD = 64
E_LOCAL = N_EXPERTS // N_DEV
ROWS_PER_DEV = E_LOCAL * CAP_PAD
N_SUB = E_LOCAL // 2
N_WBUF = 4
N_ROWS = N_EXPERTS * CAP_PAD
BLK = 256
DROP_SENTINEL = 3000


def _moe_pallas(x, rt_col, rt_row, expert_W, tok_per_dev):
    n_tokens, d_model = x.shape
    _, _, h_out = expert_W.shape
    n_blk = n_tokens // BLK

    def body(x_ref, rtc_ref, rtr_ref, w_hbm, out_ref, table, xd,
             slot_col, slot_row, wbuf, w_sems,
             send_cw, recv_cw, send_ccw, recv_ccw):
        my = lax.axis_index("i")
        left = lax.rem(my + (N_DEV - 1), N_DEV)
        right = lax.rem(my + 1, N_DEV)
        base = my * ROWS_PER_DEV
        pipe = not os.environ.get("KERNEL_NO_RING")

        barrier_sem = pltpu.get_barrier_semaphore()
        for nbr in [left, right]:
            pl.semaphore_signal(
                barrier_sem, inc=1,
                device_id=(nbr,), device_id_type=pl.DeviceIdType.MESH,
            )
        pl.semaphore_wait(barrier_sem, 2)

        def w_copy(e):
            return pltpu.make_async_copy(
                w_hbm.at[e], wbuf.at[e % N_WBUF], w_sems.at[e % N_WBUF]
            )

        for e in range(N_WBUF):
            w_copy(e).start()

        def mk_cw(h, c, s):
            r0 = c * ROWS_PER_DEV + s * CAP_PAD
            return pltpu.make_async_remote_copy(
                src_ref=table.at[pl.ds(r0, CAP_PAD)],
                dst_ref=table.at[pl.ds(r0, CAP_PAD)],
                send_sem=send_cw.at[h, s],
                recv_sem=recv_cw.at[h, s],
                device_id=(right,),
                device_id_type=pl.DeviceIdType.MESH,
            )

        def mk_ccw(h, c, s):
            r0 = c * ROWS_PER_DEV + (N_SUB + s) * CAP_PAD
            return pltpu.make_async_remote_copy(
                src_ref=table.at[pl.ds(r0, CAP_PAD)],
                dst_ref=table.at[pl.ds(r0, CAP_PAD)],
                send_sem=send_ccw.at[h, s],
                recv_sem=recv_ccw.at[h, s],
                device_id=(left,),
                device_id_type=pl.DeviceIdType.MESH,
            )

        ir = lax.broadcasted_iota(jnp.int32, (BLK, BLK), 0)
        ic = lax.broadcasted_iota(jnp.int32, (BLK, BLK), 1)
        tril = (ir >= ic).astype(jnp.bfloat16)
        triu = (ir <= ic).astype(jnp.bfloat16)

        lane32 = lax.broadcasted_iota(jnp.int32, (BLK, N_EXPERTS), 1)
        prefix = jnp.zeros((1, N_EXPERTS), jnp.float32)
        for b in range(n_blk):
            e_b = rtc_ref[b * BLK:(b + 1) * BLK, :]
            ohm = e_b == lane32
            cum_b = jnp.dot(
                tril, ohm.astype(jnp.bfloat16),
                preferred_element_type=jnp.float32,
            ) + prefix
            pos_b = jnp.sum(
                cum_b * ohm.astype(jnp.float32), axis=1, keepdims=True
            ) - 1.0
            slot_f = jnp.where(
                pos_b < float(CAP),
                e_b.astype(jnp.float32) * float(CAP_PAD) + pos_b,
                float(DROP_SENTINEL),
            )
            slot_col[b * BLK:(b + 1) * BLK, :] = slot_f.astype(jnp.int32)
            prefix = cum_b[BLK - 1:BLK, :]

        sub32 = lax.broadcasted_iota(jnp.int32, (N_EXPERTS, BLK), 0)
        prefr = jnp.zeros((N_EXPERTS, 1), jnp.float32)
        for b in range(n_blk):
            e_rb = rtr_ref[:, b * BLK:(b + 1) * BLK]
            ohm = sub32 == e_rb
            cum_rb = jnp.dot(
                ohm.astype(jnp.bfloat16), triu,
                preferred_element_type=jnp.float32,
            ) + prefr
            pos_rb = jnp.sum(
                cum_rb * ohm.astype(jnp.float32), axis=0, keepdims=True
            ) - 1.0
            slot_rf = jnp.where(
                pos_rb < float(CAP),
                e_rb.astype(jnp.float32) * float(CAP_PAD) + pos_rb,
                float(DROP_SENTINEL),
            )
            slot_row[:, b * BLK:(b + 1) * BLK] = slot_rf.astype(jnp.int32)
            prefr = cum_rb[:, BLK - 1:BLK]

        row_ids = base + lax.broadcasted_iota(
            jnp.int32, (ROWS_PER_DEV, n_tokens), 0
        )
        disp = (slot_row[...] == row_ids).astype(jnp.bfloat16)
        xd[...] = jnp.dot(
            disp, x_ref[...].astype(jnp.bfloat16),
            preferred_element_type=jnp.float32,
        ).astype(jnp.bfloat16)

        for e in range(E_LOCAL):
            w_copy(e).wait()
            res = jnp.dot(
                xd[e * CAP_PAD:(e + 1) * CAP_PAD],
                wbuf[e % N_WBUF].astype(jnp.bfloat16),
                preferred_element_type=jnp.float32,
            )
            table[pl.ds(base + e * CAP_PAD, CAP_PAD), :] = res.astype(
                jnp.bfloat16
            )
            if e + N_WBUF < E_LOCAL:
                w_copy(e + N_WBUF).start()
            if pipe:
                if e < N_SUB:
                    mk_cw(0, my, e).start()
                else:
                    mk_ccw(0, my, e - N_SUB).start()

        my_slot = slot_col[pl.ds(my * tok_per_dev, tok_per_dev), :]
        chunk_col_ids = lax.broadcasted_iota(
            jnp.int32, (tok_per_dev, ROWS_PER_DEV), 1
        )

        def combine(c, first):
            comb = (my_slot == c * ROWS_PER_DEV + chunk_col_ids).astype(
                jnp.bfloat16
            )
            part = jnp.dot(
                comb,
                table[pl.ds(c * ROWS_PER_DEV, ROWS_PER_DEV), :],
                preferred_element_type=jnp.float32,
            )
            out_ref[...] = part if first else out_ref[...] + part

        if pipe:
            combine(my, first=True)
            for h in range(1, N_DEV - 1):
                c_cw = lax.rem(my - h + N_DEV, N_DEV)
                c_ccw = lax.rem(my + h, N_DEV)
                for s in range(N_SUB):
                    mk_cw(h - 1, c_cw, s).wait_recv()
                    mk_cw(h, c_cw, s).start()
                    mk_ccw(h - 1, c_ccw, s).wait_recv()
                    mk_ccw(h, c_ccw, s).start()
                if h == N_DEV - 2:
                    combine(lax.rem(my + 2, N_DEV), first=False)
            h_last = N_DEV - 2
            c_cw = lax.rem(my + 1, N_DEV)
            c_ccw = lax.rem(my - 1 + N_DEV, N_DEV)
            for s in range(N_SUB):
                mk_cw(h_last, c_cw, s).wait_recv()
                mk_ccw(h_last, c_ccw, s).wait_recv()
            combine(c_cw, first=False)
            combine(c_ccw, first=False)
            for h in range(N_DEV - 1):
                c_cw = lax.rem(my - h + N_DEV, N_DEV)
                c_ccw = lax.rem(my + h, N_DEV)
                for s in range(N_SUB):
                    mk_cw(h, c_cw, s).wait_send()
                    mk_ccw(h, c_ccw, s).wait_send()
        else:
            combine(my, first=True)
            for d in range(1, N_DEV):
                combine(lax.rem(my + d, N_DEV), first=False)

    return pl.pallas_call(
        body,
        out_shape=jax.ShapeDtypeStruct((tok_per_dev, h_out), jnp.float32),
        in_specs=[
            pl.BlockSpec(memory_space=pltpu.VMEM),
            pl.BlockSpec(memory_space=pltpu.VMEM),
            pl.BlockSpec(memory_space=pltpu.VMEM),
            pl.BlockSpec(memory_space=pltpu.MemorySpace.HBM),
        ],
        out_specs=pl.BlockSpec(memory_space=pltpu.VMEM),
        scratch_shapes=[
            pltpu.VMEM((N_ROWS, h_out), jnp.bfloat16),
            pltpu.VMEM((ROWS_PER_DEV, d_model), jnp.bfloat16),
            pltpu.VMEM((n_tokens, 1), jnp.int32),
            pltpu.VMEM((1, n_tokens), jnp.int32),
            pltpu.VMEM((N_WBUF, d_model, h_out), jnp.float32),
            pltpu.SemaphoreType.DMA((N_WBUF,)),
            pltpu.SemaphoreType.DMA((N_DEV - 1, N_SUB)),
            pltpu.SemaphoreType.DMA((N_DEV - 1, N_SUB)),
            pltpu.SemaphoreType.DMA((N_DEV - 1, N_SUB)),
            pltpu.SemaphoreType.DMA((N_DEV - 1, N_SUB)),
        ],
        compiler_params=pltpu.CompilerParams(
            collective_id=0,
            vmem_limit_bytes=100 * 1024 * 1024,
        ),
    )(x, rt_col, rt_row, expert_W)


def kernel(x, router_W, route_idx, expert_W):
    n_tokens, _ = x.shape
    del router_W
    tok_per_dev = n_tokens // N_DEV
    rt_col = route_idx.astype(jnp.int32)
    rt_row = jnp.transpose(rt_col)
    return _moe_pallas(x, rt_col, rt_row, expert_W, tok_per_dev)
